# baseline (speedup 1.0000x reference)
import jax
import jax.numpy as jnp
from jax import lax
from jax.experimental import pallas as pl
from jax.experimental.pallas import tpu as pltpu

N = 320000
M = 10000
D = 128
B = 3200
S = 128
M_PAD = M + 3 * S + 8


def _body(idx_ref, x_ref, w_ref, pw_ref, WgT_ref, bg_ref, Wm_ref, bm_ref,
          out_ref, o_acc, d_acc):
    pid = pl.program_id(0)
    nblk = pl.num_programs(0)

    @pl.when(pid == 0)
    def _init():
        o_acc[...] = jnp.zeros_like(o_acc)
        d_acc[...] = jnp.zeros_like(d_acc)

    x = x_ref[...]
    pw = pw_ref[0, 0]
    ones_b = jnp.ones((B, 1), jnp.float32)

    idx_row = idx_ref[0]
    w_row = w_ref[0]
    gate_row = lax.dot_general(WgT_ref[...], x, (((1,), (1,)), ((), ())),
                               preferred_element_type=jnp.float32) + bg_ref[0, 0]
    a_row = jnp.exp(gate_row + pw * jnp.log(w_row))

    first = idx_ref[0, 0, 0]
    last = idx_ref[0, 0, B - 1]
    base0 = (first // 8) * 8
    nch = (last - base0) // S + 2

    iota_sb = lax.broadcasted_iota(jnp.int32, (S, B), 0)

    def chunk(c, carry):
        base = base0 + c * S
        rel = idx_row - base
        ohw = jnp.where(rel == iota_sb, a_row, 0.0)
        part_o = lax.dot_general(ohw, x, (((1,), (0,)), ((), ())),
                                 preferred_element_type=jnp.float32)
        part_d = lax.dot_general(ohw, ones_b, (((1,), (0,)), ((), ())),
                                 preferred_element_type=jnp.float32)
        o_acc[pl.ds(base, S), :] += part_o
        d_acc[pl.ds(base, S), :] += part_d
        return carry

    lax.fori_loop(0, nch, chunk, 0, unroll=False)

    @pl.when(pid == nblk - 1)
    def _finish():
        d = d_acc[:M, :]
        t = jnp.dot(o_acc[:M, :], Wm_ref[...],
                    preferred_element_type=jnp.float32)
        db = lax.dot_general(d, bm_ref[...], (((1,), (0,)), ((), ())),
                             preferred_element_type=jnp.float32)
        out_ref[...] = (t + db) / (d + 1e-10)


@jax.jit
def kernel(x, index, weights, pow_param, Wg, bg, Wm, bm):
    nblk = N // B
    idx3 = index.reshape(nblk, 1, B)
    w3 = weights.reshape(nblk, 1, B)
    grid = (nblk,)
    in_specs = [
            pl.BlockSpec((1, 1, B), lambda i: (i, 0, 0)),
            pl.BlockSpec((B, D), lambda i: (i, 0)),
            pl.BlockSpec((1, 1, B), lambda i: (i, 0, 0)),
            pl.BlockSpec((1, 1), lambda i: (0, 0)),
            pl.BlockSpec((1, D), lambda i: (0, 0)),
            pl.BlockSpec((1, 1), lambda i: (0, 0)),
            pl.BlockSpec((D, D), lambda i: (0, 0)),
            pl.BlockSpec((1, D), lambda i: (0, 0)),
        ]
    return pl.pallas_call(
        _body,
        grid=grid,
        in_specs=in_specs,
        out_specs=pl.BlockSpec((M, D), lambda i: (0, 0)),
        out_shape=jax.ShapeDtypeStruct((M, D), jnp.float32),
        scratch_shapes=[
            pltpu.VMEM((M_PAD, D), jnp.float32),
            pltpu.VMEM((M_PAD, 1), jnp.float32),
        ],
    )(idx3, x, w3, pow_param.reshape(1, 1), Wg.reshape(1, D),
      bg.reshape(1, 1), Wm, bm.reshape(1, D))


# speedup vs baseline: 1.2722x; 1.2722x over previous
import jax
import jax.numpy as jnp
from jax import lax
from jax.experimental import pallas as pl
from jax.experimental.pallas import tpu as pltpu

N = 320000
M = 10000
D = 128
B = 3200
S = 128
M_PAD = M + 3 * S + 8


def _body(idx_ref, x_ref, w_ref, pw_ref, WgT_ref, bg_ref, Wm_ref, bm_ref,
          out_ref, o_acc, d_acc):
    pid = pl.program_id(0)
    nblk = pl.num_programs(0)

    @pl.when(pid == 0)
    def _init():
        o_acc[...] = jnp.zeros_like(o_acc)
        d_acc[...] = jnp.zeros_like(d_acc)

    x = x_ref[...]
    pw = pw_ref[0, 0]
    ones_b = jnp.ones((B, 1), jnp.float32)

    idx_row = idx_ref[0]
    w_row = w_ref[0]
    gate_row = lax.dot_general(WgT_ref[...], x, (((1,), (1,)), ((), ())),
                               preferred_element_type=jnp.float32) + bg_ref[0, 0]
    a_row = jnp.exp(gate_row + pw * jnp.log(w_row))

    H = 2
    B2 = B // H
    iota_sb = lax.broadcasted_iota(jnp.int32, (S, B2), 0)
    ones_b2 = jnp.ones((B2, 1), jnp.float32)

    base0s = []
    nch_max = 0
    for h in range(H):
        firsth = idx_ref[0, 0, h * B2]
        lasth = idx_ref[0, 0, (h + 1) * B2 - 1]
        base0s.append((firsth // 8) * 8)
        nchh = (lasth - base0s[h]) // S + 1
        nch_max = nchh if h == 0 else jnp.maximum(nch_max, nchh)

    def chunk(c, carry):
        for h in range(H):
            base = base0s[h] + c * S
            rel = idx_row[:, h * B2:(h + 1) * B2] - base
            ohw = jnp.where(rel == iota_sb,
                            a_row[:, h * B2:(h + 1) * B2], 0.0)
            part_o = lax.dot_general(ohw, x[h * B2:(h + 1) * B2, :],
                                     (((1,), (0,)), ((), ())),
                                     preferred_element_type=jnp.float32)
            part_d = lax.dot_general(ohw, ones_b2, (((1,), (0,)), ((), ())),
                                     preferred_element_type=jnp.float32)
            o_acc[pl.ds(base, S), :] += part_o
            d_acc[pl.ds(base, S), :] += part_d
        return carry

    lax.fori_loop(0, nch_max, chunk, 0, unroll=False)

    @pl.when(pid == nblk - 1)
    def _finish():
        d = d_acc[:M, :]
        t = jnp.dot(o_acc[:M, :], Wm_ref[...],
                    preferred_element_type=jnp.float32)
        db = lax.dot_general(d, bm_ref[...], (((1,), (0,)), ((), ())),
                             preferred_element_type=jnp.float32)
        out_ref[...] = (t + db) / (d + 1e-10)


@jax.jit
def kernel(x, index, weights, pow_param, Wg, bg, Wm, bm):
    nblk = N // B
    idx3 = index.reshape(nblk, 1, B)
    w3 = weights.reshape(nblk, 1, B)
    grid = (nblk,)
    in_specs = [
            pl.BlockSpec((1, 1, B), lambda i: (i, 0, 0)),
            pl.BlockSpec((B, D), lambda i: (i, 0)),
            pl.BlockSpec((1, 1, B), lambda i: (i, 0, 0)),
            pl.BlockSpec((1, 1), lambda i: (0, 0)),
            pl.BlockSpec((1, D), lambda i: (0, 0)),
            pl.BlockSpec((1, 1), lambda i: (0, 0)),
            pl.BlockSpec((D, D), lambda i: (0, 0)),
            pl.BlockSpec((1, D), lambda i: (0, 0)),
        ]
    return pl.pallas_call(
        _body,
        grid=grid,
        in_specs=in_specs,
        out_specs=pl.BlockSpec((M, D), lambda i: (0, 0)),
        out_shape=jax.ShapeDtypeStruct((M, D), jnp.float32),
        scratch_shapes=[
            pltpu.VMEM((M_PAD, D), jnp.float32),
            pltpu.VMEM((M_PAD, 1), jnp.float32),
        ],
    )(idx3, x, w3, pow_param.reshape(1, 1), Wg.reshape(1, D),
      bg.reshape(1, 1), Wm, bm.reshape(1, D))


# speedup vs baseline: 1.2793x; 1.0056x over previous
import jax
import jax.numpy as jnp
from jax import lax
from jax.experimental import pallas as pl
from jax.experimental.pallas import tpu as pltpu

N = 320000
M = 10000
D = 128
B = 3200
S = 128
M_PAD = M + 3 * S + 8


def _body(idx_ref, x_ref, w_ref, pw_ref, WgT_ref, bg_ref, Wm_ref, bm_ref,
          out_ref, o_acc, d_acc):
    pid = pl.program_id(0)
    nblk = pl.num_programs(0)

    @pl.when(pid == 0)
    def _init():
        o_acc[...] = jnp.zeros_like(o_acc)
        d_acc[...] = jnp.zeros_like(d_acc)

    x = x_ref[...]
    pw = pw_ref[0, 0]
    ones_b = jnp.ones((B, 1), jnp.float32)

    idx_row = idx_ref[0]
    w_row = w_ref[0]
    gate_row = lax.dot_general(WgT_ref[...], x, (((1,), (1,)), ((), ())),
                               preferred_element_type=jnp.float32) + bg_ref[0, 0]
    a_row = jnp.exp(gate_row + pw * jnp.log(w_row))

    first = idx_ref[0, 0, 0]
    last = idx_ref[0, 0, B - 1]
    base0 = (first // 8) * 8
    nch = (last - base0) // S + 1

    iota_sb = lax.broadcasted_iota(jnp.int32, (S, B), 0)

    def chunk(c, carry):
        base = base0 + c * S
        rel = idx_row - base
        ohw = jnp.where(rel == iota_sb, a_row, 0.0)
        part_o = lax.dot_general(ohw, x, (((1,), (0,)), ((), ())),
                                 preferred_element_type=jnp.float32)
        part_d = lax.dot_general(ohw, ones_b, (((1,), (0,)), ((), ())),
                                 preferred_element_type=jnp.float32)
        o_acc[pl.ds(base, S), :] += part_o
        d_acc[pl.ds(base, S), :] += part_d
        return carry

    lax.fori_loop(0, nch, chunk, 0, unroll=False)

    @pl.when(pid == nblk - 1)
    def _finish():
        d = d_acc[:M, :]
        t = jnp.dot(o_acc[:M, :], Wm_ref[...],
                    preferred_element_type=jnp.float32)
        db = lax.dot_general(d, bm_ref[...], (((1,), (0,)), ((), ())),
                             preferred_element_type=jnp.float32)
        out_ref[...] = (t + db) / (d + 1e-10)


@jax.jit
def kernel(x, index, weights, pow_param, Wg, bg, Wm, bm):
    nblk = N // B
    idx3 = index.reshape(nblk, 1, B)
    w3 = weights.reshape(nblk, 1, B)
    grid = (nblk,)
    in_specs = [
            pl.BlockSpec((1, 1, B), lambda i: (i, 0, 0)),
            pl.BlockSpec((B, D), lambda i: (i, 0)),
            pl.BlockSpec((1, 1, B), lambda i: (i, 0, 0)),
            pl.BlockSpec((1, 1), lambda i: (0, 0)),
            pl.BlockSpec((1, D), lambda i: (0, 0)),
            pl.BlockSpec((1, 1), lambda i: (0, 0)),
            pl.BlockSpec((D, D), lambda i: (0, 0)),
            pl.BlockSpec((1, D), lambda i: (0, 0)),
        ]
    return pl.pallas_call(
        _body,
        grid=grid,
        in_specs=in_specs,
        out_specs=pl.BlockSpec((M, D), lambda i: (0, 0)),
        out_shape=jax.ShapeDtypeStruct((M, D), jnp.float32),
        scratch_shapes=[
            pltpu.VMEM((M_PAD, D), jnp.float32),
            pltpu.VMEM((M_PAD, 1), jnp.float32),
        ],
    )(idx3, x, w3, pow_param.reshape(1, 1), Wg.reshape(1, D),
      bg.reshape(1, 1), Wm, bm.reshape(1, D))
